# baseline (speedup 1.0000x reference)
import functools

import jax
import jax.numpy as jnp
from jax import lax
from jax.experimental import pallas as pl
from jax.experimental.pallas import tpu as pltpu
from jax.experimental.pallas import tpu_sc as plsc

B, D = 16384, 128
S = 4096
NC, NS = 2, 16
NW = NC * NS
ROWS_W = S // NW
CH = 128
NCH = ROWS_W // CH
RANK = D // 2
LANES = 16
G = RANK // LANES

_mesh = plsc.VectorSubcoreMesh(core_axis_name="c", subcore_axis_name="s")


@functools.partial(
    pl.kernel,
    out_type=jax.ShapeDtypeStruct((S, D), jnp.float32),
    mesh=_mesh,
    scratch_types=[
        pltpu.VMEM((CH, D), jnp.float32),
        pltpu.VMEM((CH, D), jnp.float32),
        pltpu.VMEM((CH, D), jnp.float32),
        pltpu.VMEM((CH, D), jnp.float32),
        pltpu.VMEM((CH, D), jnp.float32),
        pltpu.VMEM((CH, D), jnp.float32),
        pltpu.SemaphoreType.DMA,
        pltpu.SemaphoreType.DMA,
        pltpu.SemaphoreType.DMA,
        pltpu.SemaphoreType.DMA,
        pltpu.SemaphoreType.DMA,
        pltpu.SemaphoreType.DMA,
    ],
)
def _sc_complex(lhs_hbm, rel_hbm, out_hbm, lv0, lv1, rv0, rv1, ov0, ov1,
                sl0, sl1, sr0, sr1, so0, so1):
    lv, rv, ov = [lv0, lv1], [rv0, rv1], [ov0, ov1]
    sl, sr, so = [sl0, sl1], [sr0, sr1], [so0, so1]

    wid = lax.axis_index("s") * NC + lax.axis_index("c")
    base = wid * ROWS_W

    def start_in(ci):
        b = ci % 2
        row0 = base + ci * CH
        cl = pltpu.make_async_copy(lhs_hbm.at[pl.ds(row0, CH)], lv[b], sl[b])
        cr = pltpu.make_async_copy(rel_hbm.at[pl.ds(row0, CH)], rv[b], sr[b])
        cl.start()
        cr.start()
        return cl, cr

    def start_out(ci):
        b = ci % 2
        row0 = base + ci * CH
        co = pltpu.make_async_copy(ov[b], out_hbm.at[pl.ds(row0, CH)], so[b])
        co.start()
        return co

    def compute(lhs_v, rel_v, out_v):
        def rowbody(i, c2):
            for g in range(G):
                lo = g * LANES
                hi = RANK + g * LANES
                l0 = lhs_v[i, pl.ds(lo, LANES)]
                l1 = lhs_v[i, pl.ds(hi, LANES)]
                r0 = rel_v[i, pl.ds(lo, LANES)]
                r1 = rel_v[i, pl.ds(hi, LANES)]
                out_v[i, pl.ds(lo, LANES)] = l0 * r0 - l1 * r1
                out_v[i, pl.ds(hi, LANES)] = l0 * r1 + l1 * r0
            return c2

        lax.fori_loop(0, CH, rowbody, 0)

    pend_in = {0: start_in(0)}
    pend_out = {}
    for ci in range(NCH):
        if ci + 1 < NCH:
            pend_in[ci + 1] = start_in(ci + 1)
        cl, cr = pend_in.pop(ci)
        cl.wait()
        cr.wait()
        if ci - 2 in pend_out:
            pend_out.pop(ci - 2).wait()
        compute(lv[ci % 2], rv[ci % 2], ov[ci % 2])
        pend_out[ci] = start_out(ci)
    for co in pend_out.values():
        co.wait()


def _tc_body(lhs_ref, rel_ref, out_ref):
    lhs = lhs_ref[...]
    rel = rel_ref[...]
    r = lhs.shape[-1] // 2
    l0, l1 = lhs[:, :r], lhs[:, r:]
    r0, r1 = rel[:, :r], rel[:, r:]
    out_ref[:, :r] = l0 * r0 - l1 * r1
    out_ref[:, r:] = l0 * r1 + l1 * r0


def _tc_complex(lhs, rel):
    n, d = lhs.shape
    blk = 2048
    return pl.pallas_call(
        _tc_body,
        grid=(n // blk,),
        in_specs=[
            pl.BlockSpec((blk, d), lambda i: (i, 0)),
            pl.BlockSpec((blk, d), lambda i: (i, 0)),
        ],
        out_specs=pl.BlockSpec((blk, d), lambda i: (i, 0)),
        out_shape=jax.ShapeDtypeStruct((n, d), lhs.dtype),
    )(lhs, rel)


def kernel(lhs, rel):
    sc_part = _sc_complex(lhs[:S], rel[:S])
    tc_part = _tc_complex(lhs[S:], rel[S:])
    return jnp.concatenate([sc_part, tc_part], axis=0)


# speedup vs baseline: 1.5943x; 1.5943x over previous
import functools

import jax
import jax.numpy as jnp
from jax import lax
from jax.experimental import pallas as pl
from jax.experimental.pallas import tpu as pltpu
from jax.experimental.pallas import tpu_sc as plsc

B, D = 16384, 128
S = 4096
NC, NS = 2, 16
NW = NC * NS
ROWS_W = S // NW
CH = 128
NCH = ROWS_W // CH
RANK = D // 2
LANES = 16
G = RANK // LANES

_mesh = plsc.VectorSubcoreMesh(core_axis_name="c", subcore_axis_name="s")


@functools.partial(
    pl.kernel,
    out_type=jax.ShapeDtypeStruct((S, D), jnp.float32),
    mesh=_mesh,
    scratch_types=[
        pltpu.VMEM((CH, D), jnp.float32),
        pltpu.VMEM((CH, D), jnp.float32),
        pltpu.VMEM((CH, D), jnp.float32),
        pltpu.VMEM((CH, D), jnp.float32),
        pltpu.VMEM((CH, D), jnp.float32),
        pltpu.VMEM((CH, D), jnp.float32),
        pltpu.SemaphoreType.DMA,
        pltpu.SemaphoreType.DMA,
        pltpu.SemaphoreType.DMA,
        pltpu.SemaphoreType.DMA,
        pltpu.SemaphoreType.DMA,
        pltpu.SemaphoreType.DMA,
    ],
)
def _sc_complex(lhs_hbm, rel_hbm, out_hbm, lv0, lv1, rv0, rv1, ov0, ov1,
                sl0, sl1, sr0, sr1, so0, so1):
    lv, rv, ov = [lv0, lv1], [rv0, rv1], [ov0, ov1]
    sl, sr, so = [sl0, sl1], [sr0, sr1], [so0, so1]

    wid = lax.axis_index("s") * NC + lax.axis_index("c")
    base = wid * ROWS_W

    def start_in(ci):
        b = ci % 2
        row0 = base + ci * CH
        cl = pltpu.make_async_copy(lhs_hbm.at[pl.ds(row0, CH)], lv[b], sl[b])
        cr = pltpu.make_async_copy(rel_hbm.at[pl.ds(row0, CH)], rv[b], sr[b])
        cl.start()
        cr.start()
        return cl, cr

    def start_out(ci):
        b = ci % 2
        row0 = base + ci * CH
        co = pltpu.make_async_copy(ov[b], out_hbm.at[pl.ds(row0, CH)], so[b])
        co.start()
        return co

    def compute(lhs_v, rel_v, out_v):
        def rowbody(i, c2):
            for g in range(G):
                lo = g * LANES
                hi = RANK + g * LANES
                l0 = lhs_v[i, pl.ds(lo, LANES)]
                l1 = lhs_v[i, pl.ds(hi, LANES)]
                r0 = rel_v[i, pl.ds(lo, LANES)]
                r1 = rel_v[i, pl.ds(hi, LANES)]
                out_v[i, pl.ds(lo, LANES)] = l0 * r0 - l1 * r1
                out_v[i, pl.ds(hi, LANES)] = l0 * r1 + l1 * r0
            return c2

        lax.fori_loop(0, CH, rowbody, 0)

    pend_in = {0: start_in(0)}
    pend_out = {}
    for ci in range(NCH):
        if ci + 1 < NCH:
            pend_in[ci + 1] = start_in(ci + 1)
        cl, cr = pend_in.pop(ci)
        cl.wait()
        cr.wait()
        if ci - 2 in pend_out:
            pend_out.pop(ci - 2).wait()
        compute(lv[ci % 2], rv[ci % 2], ov[ci % 2])
        pend_out[ci] = start_out(ci)
    for co in pend_out.values():
        co.wait()


def _tc_body(lhs_ref, rel_ref, out_ref):
    lhs = lhs_ref[...]
    rel = rel_ref[...]
    r = lhs.shape[-1] // 2
    l0, l1 = lhs[:, :r], lhs[:, r:]
    r0, r1 = rel[:, :r], rel[:, r:]
    out_ref[:, :r] = l0 * r0 - l1 * r1
    out_ref[:, r:] = l0 * r1 + l1 * r0


def _tc_complex(lhs, rel):
    n, d = lhs.shape
    blk = 2048
    off = S // blk
    return pl.pallas_call(
        _tc_body,
        grid=((n - S) // blk,),
        in_specs=[
            pl.BlockSpec((blk, d), lambda i: (i + off, 0)),
            pl.BlockSpec((blk, d), lambda i: (i + off, 0)),
        ],
        out_specs=pl.BlockSpec((blk, d), lambda i: (i + off, 0)),
        out_shape=jax.ShapeDtypeStruct((n, d), lhs.dtype),
    )(lhs, rel)


def kernel(lhs, rel):
    sc_part = _sc_complex(lhs, rel)
    tc_out = _tc_complex(lhs, rel)
    return lax.dynamic_update_slice(tc_out, sc_part, (0, 0))


# speedup vs baseline: 3.9191x; 2.4582x over previous
import jax
import jax.numpy as jnp
from jax import lax
from jax.experimental import pallas as pl
from jax.experimental.pallas import tpu as pltpu


def _complex_body(lhs_ref, rel_ref, out_ref):
    lhs = lhs_ref[...]
    rel = rel_ref[...]
    n, d = lhs.shape
    r = d // 2
    col = lax.broadcasted_iota(jnp.int32, (n, d), 1)
    first = col < r
    rrel = pltpu.roll(rel, r, 1)
    rlhs = pltpu.roll(lhs, r, 1)
    a = jnp.where(first, rel, rrel)
    b = jnp.where(first, -rrel, rel)
    out_ref[...] = lhs * a + rlhs * b


def kernel(lhs, rel):
    B, D = lhs.shape
    blk = 2048
    return pl.pallas_call(
        _complex_body,
        grid=(B // blk,),
        in_specs=[
            pl.BlockSpec((blk, D), lambda i: (i, 0)),
            pl.BlockSpec((blk, D), lambda i: (i, 0)),
        ],
        out_specs=pl.BlockSpec((blk, D), lambda i: (i, 0)),
        out_shape=jax.ShapeDtypeStruct((B, D), lhs.dtype),
    )(lhs, rel)


# speedup vs baseline: 4.4941x; 1.1467x over previous
import jax
import jax.numpy as jnp
from jax import lax
from jax.experimental import pallas as pl
from jax.experimental.pallas import tpu as pltpu


def _complex_body(lhs_ref, rel_ref, out_ref):
    lhs = lhs_ref[...]
    rel = rel_ref[...]
    n, d = lhs.shape
    r = d // 2
    col = lax.broadcasted_iota(jnp.int32, (n, d), 1)
    first = col < r
    rrel = pltpu.roll(rel, r, 1)
    rlhs = pltpu.roll(lhs, r, 1)
    a = jnp.where(first, rel, rrel)
    b = jnp.where(first, -rrel, rel)
    out_ref[...] = lhs * a + rlhs * b


def kernel(lhs, rel):
    B, D = lhs.shape
    blk = 4096
    return pl.pallas_call(
        _complex_body,
        grid=(B // blk,),
        in_specs=[
            pl.BlockSpec((blk, D), lambda i: (i, 0)),
            pl.BlockSpec((blk, D), lambda i: (i, 0)),
        ],
        out_specs=pl.BlockSpec((blk, D), lambda i: (i, 0)),
        out_shape=jax.ShapeDtypeStruct((B, D), lhs.dtype),
    )(lhs, rel)


# speedup vs baseline: 4.8632x; 1.0821x over previous
import jax
import jax.numpy as jnp
from jax import lax
from jax.experimental import pallas as pl
from jax.experimental.pallas import tpu as pltpu


def _complex_body(lhs_ref, rel_ref, out_ref):
    lhs = lhs_ref[...]
    rel = rel_ref[...]
    n, d = lhs.shape
    r = d // 2
    col = lax.broadcasted_iota(jnp.int32, (n, d), 1)
    first = col < r
    rrel = pltpu.roll(rel, r, 1)
    rlhs = pltpu.roll(lhs, r, 1)
    a = jnp.where(first, rel, rrel)
    b = jnp.where(first, -rrel, rel)
    out_ref[...] = lhs * a + rlhs * b


def kernel(lhs, rel):
    B, D = lhs.shape
    blk = 8192
    return pl.pallas_call(
        _complex_body,
        grid=(B // blk,),
        in_specs=[
            pl.BlockSpec((blk, D), lambda i: (i, 0)),
            pl.BlockSpec((blk, D), lambda i: (i, 0)),
        ],
        out_specs=pl.BlockSpec((blk, D), lambda i: (i, 0)),
        out_shape=jax.ShapeDtypeStruct((B, D), lhs.dtype),
    )(lhs, rel)
